# baseline (speedup 1.0000x reference)
import jax
import jax.numpy as jnp
from jax.experimental import pallas as pl
from jax.experimental.pallas import tpu as pltpu
from jax.experimental.pallas import tpu_sc as plsc

_N = 2048
_E = 32768
_FILTER = 5
_SUBCORES = 16
_CORES = 2


def _build_adj(edge_index):
    nn_tot = _N * _N
    n_sub = _CORES * _SUBCORES
    rows_per_sub = nn_tot // n_sub
    chunk = _E // _SUBCORES
    zrows = 32768

    flat = (edge_index[1].astype(jnp.int32) * _N
            + edge_index[0].astype(jnp.int32))
    ones = jnp.ones((_E,), jnp.float32)

    mesh = plsc.VectorSubcoreMesh(core_axis_name="c", subcore_axis_name="s")

    @pl.kernel(
        out_type=jax.ShapeDtypeStruct((nn_tot + 8,), jnp.float32),
        mesh=mesh,
        scratch_types=[
            pltpu.VMEM((chunk,), jnp.int32),
            pltpu.VMEM((chunk,), jnp.int32),
            pltpu.VMEM((chunk,), jnp.float32),
            pltpu.VMEM((zrows,), jnp.float32),
        ],
    )
    def sc_build(flat_hbm, ones_hbm, o_hbm, idx_in, idx_out, vals, zbuf):
        c = jax.lax.axis_index("c")
        s = jax.lax.axis_index("s")
        gsid = c * _SUBCORES + s

        @pl.loop(0, zrows, step=16)
        def _(j):
            zbuf[pl.ds(j, 16)] = jnp.zeros((16,), jnp.float32)

        base = gsid * rows_per_sub

        @pl.loop(0, rows_per_sub, step=zrows)
        def _(r):
            pltpu.sync_copy(zbuf, o_hbm.at[pl.ds(base + r, zrows)])

        plsc.subcore_barrier()

        off = s * chunk
        pltpu.sync_copy(flat_hbm.at[pl.ds(off, chunk)], idx_in)
        pltpu.sync_copy(ones_hbm.at[pl.ds(off, chunk)], vals)

        half = nn_tot // _CORES
        lo = c * half
        hi = lo + half

        @pl.loop(0, chunk, step=16)
        def _(j):
            v = idx_in[pl.ds(j, 16)]
            ok = (v >= lo) & (v < hi)
            idx_out[pl.ds(j, 16)] = jnp.where(ok, v, nn_tot)

        pltpu.sync_copy(vals, o_hbm.at[idx_out])

    out = sc_build(flat, ones)
    return out[:nn_tot].reshape(_N, _N)


def _mm(a, b, bm, bn, bk, lhs_scale=None, rhs_scale=None, out_scale=None,
        bias=None, relu=False, precision=jax.lax.Precision.DEFAULT):
    m, k = a.shape
    k2, n = b.shape
    assert k == k2 and m % bm == 0 and n % bn == 0 and k % bk == 0
    nm, nn, nk = m // bm, n // bn, k // bk

    operands = [a, b]
    in_specs = [
        pl.BlockSpec((bm, bk), lambda mi, ni, ki: (mi, ki)),
        pl.BlockSpec((bk, bn), lambda mi, ni, ki: (ki, ni)),
    ]
    if lhs_scale is not None:
        operands.append(lhs_scale)
        in_specs.append(pl.BlockSpec((bm, 1), lambda mi, ni, ki: (mi, 0)))
    if rhs_scale is not None:
        operands.append(rhs_scale)
        in_specs.append(pl.BlockSpec((bk, 1), lambda mi, ni, ki: (ki, 0)))
    if out_scale is not None:
        operands.append(out_scale)
        in_specs.append(pl.BlockSpec((bm, 1), lambda mi, ni, ki: (mi, 0)))
    if bias is not None:
        operands.append(bias.reshape(1, n))
        in_specs.append(pl.BlockSpec((1, bn), lambda mi, ni, ki: (0, ni)))

    def body(a_ref, b_ref, *rest):
        rest = list(rest)
        ls_ref = rest.pop(0) if lhs_scale is not None else None
        rs_ref = rest.pop(0) if rhs_scale is not None else None
        os_ref = rest.pop(0) if out_scale is not None else None
        bi_ref = rest.pop(0) if bias is not None else None
        o_ref = rest.pop(0)

        av = a_ref[...]
        if ls_ref is not None:
            av = av * ls_ref[...]
        bv = b_ref[...]
        if rs_ref is not None:
            bv = bv * rs_ref[...]
        part = jnp.dot(av, bv, preferred_element_type=jnp.float32,
                       precision=precision)

        def epilogue(r):
            if os_ref is not None:
                r = r * os_ref[...]
            if bi_ref is not None:
                r = r + bi_ref[...]
            if relu:
                r = jnp.maximum(r, 0.0)
            return r

        if nk == 1:
            o_ref[...] = epilogue(part)
        else:
            ki = pl.program_id(2)

            @pl.when(ki == 0)
            def _():
                o_ref[...] = part

            @pl.when(ki > 0)
            def _():
                o_ref[...] += part

            if out_scale is not None or bias is not None or relu:
                @pl.when(ki == nk - 1)
                def _():
                    o_ref[...] = epilogue(o_ref[...])

    return pl.pallas_call(
        body,
        grid=(nm, nn, nk),
        in_specs=in_specs,
        out_specs=pl.BlockSpec((bm, bn), lambda mi, ni, ki: (mi, ni)),
        out_shape=jax.ShapeDtypeStruct((m, n), jnp.float32),
        compiler_params=pltpu.CompilerParams(
            dimension_semantics=("parallel", "parallel", "arbitrary")),
    )(*operands)


def _combine(p1, p2, p3, p4, p5, ws):
    br = 256
    ng = _N // br

    def body(w_ref, p1_ref, p2_ref, p3_ref, p4_ref, p5_ref,
             m1_ref, m2_ref, m3_ref, d1_ref, d2_ref, d3_ref):
        i = pl.program_id(0)
        rows = jax.lax.broadcasted_iota(jnp.int32, (br, _N), 0) + i * br
        cols = jax.lax.broadcasted_iota(jnp.int32, (br, _N), 1)
        diag = rows == cols
        pv = (p1_ref[...], p2_ref[...], p3_ref[...], p4_ref[...], p5_ref[...])
        for l, (m_ref, d_ref) in enumerate(
                ((m1_ref, d1_ref), (m2_ref, d2_ref), (m3_ref, d3_ref))):
            c = w_ref[l, 0]
            mv = jnp.where(diag, c, 0.0)
            for i_pow in range(_FILTER):
                c = c * w_ref[l, i_pow + 1]
                mv = mv + c * pv[i_pow]
            m_ref[...] = mv
            deg = jnp.sum((mv != 0.0).astype(jnp.float32), axis=1,
                          keepdims=True)
            d_ref[...] = jnp.where(
                deg > 0.0, jax.lax.rsqrt(jnp.maximum(deg, 1.0)), 0.0)

    pspec = pl.BlockSpec((br, _N), lambda i: (i, 0))
    dspec = pl.BlockSpec((br, 1), lambda i: (i, 0))
    mshape = jax.ShapeDtypeStruct((_N, _N), jnp.float32)
    dshape = jax.ShapeDtypeStruct((_N, 1), jnp.float32)
    return pl.pallas_call(
        body,
        grid=(ng,),
        in_specs=[pl.BlockSpec(memory_space=pltpu.SMEM)] + [pspec] * 5,
        out_specs=[pspec] * 3 + [dspec] * 3,
        out_shape=[mshape] * 3 + [dshape] * 3,
        compiler_params=pltpu.CompilerParams(
            dimension_semantics=("arbitrary",)),
    )(ws, p1, p2, p3, p4, p5)


def kernel(x, edge_index, W1, b1, w1, W2, b2, w2, W3, b3, w3):
    A = _build_adj(edge_index)

    p2 = _mm(A, A, 1024, 1024, 1024)
    p3 = _mm(p2, A, 1024, 1024, 1024)
    p4 = _mm(p3, A, 1024, 1024, 1024)
    p5 = _mm(p4, A, 1024, 1024, 1024)

    m1, m2, m3, d1, d2, d3 = _combine(A, p2, p3, p4, p5,
                                      jnp.stack([w1, w2, w3]))

    t = _mm(m1, x, 1024, 128, 2048, rhs_scale=d1)
    h = _mm(t, W1, 2048, 1280, 128, lhs_scale=d1, bias=b1, relu=True)

    u = _mm(h, W2, 1024, 3200, 256)
    h = _mm(m2, u, 1024, 3200, 256, rhs_scale=d2, out_scale=d2,
            bias=b2, relu=True)

    u = _mm(h, W3, 2048, 64, 640)
    out = _mm(m3, u, 1024, 64, 2048, rhs_scale=d3, out_scale=d3,
              bias=b3, relu=True)
    return out


# speedup vs baseline: 9.8906x; 9.8906x over previous
import dataclasses

import jax
import jax.numpy as jnp
from jax.experimental import pallas as pl
from jax.experimental.pallas import tpu as pltpu
from jax.experimental.pallas import tpu_sc as plsc

_N = 2048
_E = 32768
_FILTER = 5
_SUBCORES = 16
_CORES = 2


def _build_adj(edge_index):
    nn_tot = _N * _N
    n_sub = _CORES * _SUBCORES
    n_pass = 2
    tile_elems = nn_tot // (n_sub * n_pass)

    flat = (edge_index[1].astype(jnp.int32) * _N
            + edge_index[0].astype(jnp.int32))

    mesh = plsc.VectorSubcoreMesh(core_axis_name="c", subcore_axis_name="s")

    @pl.kernel(
        out_type=jax.ShapeDtypeStruct((nn_tot,), jnp.float32),
        mesh=mesh,
        scratch_types=[
            pltpu.VMEM((_E,), jnp.int32),
            pltpu.VMEM((tile_elems,), jnp.float32),
        ],
        compiler_params=dataclasses.replace(
            pltpu.CompilerParams(), needs_layout_passes=False),
    )
    def sc_build(flat_hbm, o_hbm, edges, tile):
        c = jax.lax.axis_index("c")
        s = jax.lax.axis_index("s")
        gsid = c * _SUBCORES + s

        pltpu.sync_copy(flat_hbm, edges)
        one16 = jnp.ones((16,), jnp.float32)

        for p in range(n_pass):
            base = (gsid * n_pass + p) * tile_elems

            @pl.loop(0, tile_elems, step=128)
            def _(j):
                for u in range(0, 128, 16):
                    tile[pl.ds(j + u, 16)] = jnp.zeros((16,), jnp.float32)

            @pl.loop(0, _E, step=64)
            def _(j):
                for u in range(0, 64, 16):
                    v = edges[pl.ds(j + u, 16)]
                    loc = v - base
                    ok = (loc >= 0) & (loc < tile_elems)
                    locc = jnp.where(ok, loc, 0)
                    plsc.store_scatter(tile, [locc], one16, mask=ok)

            pltpu.sync_copy(tile, o_hbm.at[pl.ds(base, tile_elems)])

    out = sc_build(flat)
    return out.reshape(_N, _N)


def _mm(a, b, bm, bn, bk, lhs_scale=None, rhs_scale=None, out_scale=None,
        bias=None, relu=False, precision=jax.lax.Precision.DEFAULT):
    m, k = a.shape
    k2, n = b.shape
    assert k == k2 and m % bm == 0 and n % bn == 0 and k % bk == 0
    nm, nn, nk = m // bm, n // bn, k // bk

    operands = [a, b]
    in_specs = [
        pl.BlockSpec((bm, bk), lambda mi, ni, ki: (mi, ki)),
        pl.BlockSpec((bk, bn), lambda mi, ni, ki: (ki, ni)),
    ]
    if lhs_scale is not None:
        operands.append(lhs_scale)
        in_specs.append(pl.BlockSpec((bm, 1), lambda mi, ni, ki: (mi, 0)))
    if rhs_scale is not None:
        operands.append(rhs_scale)
        in_specs.append(pl.BlockSpec((bk, 1), lambda mi, ni, ki: (ki, 0)))
    if out_scale is not None:
        operands.append(out_scale)
        in_specs.append(pl.BlockSpec((bm, 1), lambda mi, ni, ki: (mi, 0)))
    if bias is not None:
        operands.append(bias.reshape(1, n))
        in_specs.append(pl.BlockSpec((1, bn), lambda mi, ni, ki: (0, ni)))

    def body(a_ref, b_ref, *rest):
        rest = list(rest)
        ls_ref = rest.pop(0) if lhs_scale is not None else None
        rs_ref = rest.pop(0) if rhs_scale is not None else None
        os_ref = rest.pop(0) if out_scale is not None else None
        bi_ref = rest.pop(0) if bias is not None else None
        o_ref = rest.pop(0)

        av = a_ref[...]
        if ls_ref is not None:
            av = av * ls_ref[...]
        bv = b_ref[...]
        if rs_ref is not None:
            bv = bv * rs_ref[...]
        part = jnp.dot(av, bv, preferred_element_type=jnp.float32,
                       precision=precision)

        def epilogue(r):
            if os_ref is not None:
                r = r * os_ref[...]
            if bi_ref is not None:
                r = r + bi_ref[...]
            if relu:
                r = jnp.maximum(r, 0.0)
            return r

        if nk == 1:
            o_ref[...] = epilogue(part)
        else:
            ki = pl.program_id(2)

            @pl.when(ki == 0)
            def _():
                o_ref[...] = part

            @pl.when(ki > 0)
            def _():
                o_ref[...] += part

            if out_scale is not None or bias is not None or relu:
                @pl.when(ki == nk - 1)
                def _():
                    o_ref[...] = epilogue(o_ref[...])

    return pl.pallas_call(
        body,
        grid=(nm, nn, nk),
        in_specs=in_specs,
        out_specs=pl.BlockSpec((bm, bn), lambda mi, ni, ki: (mi, ni)),
        out_shape=jax.ShapeDtypeStruct((m, n), jnp.float32),
        compiler_params=pltpu.CompilerParams(
            dimension_semantics=("parallel", "parallel", "arbitrary")),
    )(*operands)


def _combine(p1, p2, p3, p4, p5, ws):
    br = 256
    ng = _N // br

    def body(w_ref, p1_ref, p2_ref, p3_ref, p4_ref, p5_ref,
             m1_ref, m2_ref, m3_ref, d1_ref, d2_ref, d3_ref):
        i = pl.program_id(0)
        rows = jax.lax.broadcasted_iota(jnp.int32, (br, _N), 0) + i * br
        cols = jax.lax.broadcasted_iota(jnp.int32, (br, _N), 1)
        diag = rows == cols
        pv = (p1_ref[...], p2_ref[...], p3_ref[...], p4_ref[...], p5_ref[...])
        for l, (m_ref, d_ref) in enumerate(
                ((m1_ref, d1_ref), (m2_ref, d2_ref), (m3_ref, d3_ref))):
            c = w_ref[l, 0]
            mv = jnp.where(diag, c, 0.0)
            for i_pow in range(_FILTER):
                c = c * w_ref[l, i_pow + 1]
                mv = mv + c * pv[i_pow]
            m_ref[...] = mv
            deg = jnp.sum((mv != 0.0).astype(jnp.float32), axis=1,
                          keepdims=True)
            d_ref[...] = jnp.where(
                deg > 0.0, jax.lax.rsqrt(jnp.maximum(deg, 1.0)), 0.0)

    pspec = pl.BlockSpec((br, _N), lambda i: (i, 0))
    dspec = pl.BlockSpec((br, 1), lambda i: (i, 0))
    mshape = jax.ShapeDtypeStruct((_N, _N), jnp.float32)
    dshape = jax.ShapeDtypeStruct((_N, 1), jnp.float32)
    return pl.pallas_call(
        body,
        grid=(ng,),
        in_specs=[pl.BlockSpec(memory_space=pltpu.SMEM)] + [pspec] * 5,
        out_specs=[pspec] * 3 + [dspec] * 3,
        out_shape=[mshape] * 3 + [dshape] * 3,
        compiler_params=pltpu.CompilerParams(
            dimension_semantics=("arbitrary",)),
    )(ws, p1, p2, p3, p4, p5)


def kernel(x, edge_index, W1, b1, w1, W2, b2, w2, W3, b3, w3):
    A = _build_adj(edge_index)

    p2 = _mm(A, A, 1024, 1024, 1024)
    p3 = _mm(p2, A, 1024, 1024, 1024)
    p4 = _mm(p3, A, 1024, 1024, 1024)
    p5 = _mm(p4, A, 1024, 1024, 1024)

    m1, m2, m3, d1, d2, d3 = _combine(A, p2, p3, p4, p5,
                                      jnp.stack([w1, w2, w3]))

    t = _mm(m1, x, 1024, 128, 2048, rhs_scale=d1)
    h = _mm(t, W1, 2048, 1280, 128, lhs_scale=d1, bias=b1, relu=True)

    u = _mm(h, W2, 1024, 3200, 256)
    h = _mm(m2, u, 1024, 3200, 256, rhs_scale=d2, out_scale=d2,
            bias=b2, relu=True)

    u = _mm(h, W3, 2048, 64, 640)
    out = _mm(m3, u, 1024, 64, 2048, rhs_scale=d3, out_scale=d3,
              bias=b3, relu=True)
    return out


# speedup vs baseline: 13.4528x; 1.3602x over previous
import dataclasses

import jax
import jax.numpy as jnp
from jax.experimental import pallas as pl
from jax.experimental.pallas import tpu as pltpu
from jax.experimental.pallas import tpu_sc as plsc

_N = 2048
_E = 32768
_FILTER = 5
_SUBCORES = 16
_CORES = 2


def _build_adj(edge_index):
    nn_tot = _N * _N
    n_sub = _CORES * _SUBCORES
    n_pass = 2
    tile_elems = nn_tot // (n_sub * n_pass)

    flat = (edge_index[1].astype(jnp.int32) * _N
            + edge_index[0].astype(jnp.int32))

    mesh = plsc.VectorSubcoreMesh(core_axis_name="c", subcore_axis_name="s")

    @pl.kernel(
        out_type=jax.ShapeDtypeStruct((nn_tot,), jnp.float32),
        mesh=mesh,
        scratch_types=[
            pltpu.VMEM((_E,), jnp.int32),
            pltpu.VMEM((tile_elems,), jnp.float32),
        ],
        compiler_params=dataclasses.replace(
            pltpu.CompilerParams(), needs_layout_passes=False),
    )
    def sc_build(flat_hbm, o_hbm, edges, tile):
        c = jax.lax.axis_index("c")
        s = jax.lax.axis_index("s")
        gsid = c * _SUBCORES + s

        pltpu.sync_copy(flat_hbm, edges)
        one16 = jnp.ones((16,), jnp.float32)

        for p in range(n_pass):
            base = (gsid * n_pass + p) * tile_elems

            @pl.loop(0, tile_elems, step=128)
            def _(j):
                for u in range(0, 128, 16):
                    tile[pl.ds(j + u, 16)] = jnp.zeros((16,), jnp.float32)

            @pl.loop(0, _E, step=64)
            def _(j):
                for u in range(0, 64, 16):
                    v = edges[pl.ds(j + u, 16)]
                    loc = v - base
                    ok = (loc >= 0) & (loc < tile_elems)
                    locc = jnp.where(ok, loc, 0)
                    plsc.store_scatter(tile, [locc], one16, mask=ok)

            pltpu.sync_copy(tile, o_hbm.at[pl.ds(base, tile_elems)])

    out = sc_build(flat)
    return out.reshape(_N, _N)


def _mm(a, b, bm, bn, bk, lhs_scale=None, rhs_scale=None, out_scale=None,
        bias=None, relu=False, out_dtype=jnp.bfloat16):
    m, k = a.shape
    k2, n = b.shape
    assert k == k2 and m % bm == 0 and n % bn == 0 and k % bk == 0
    nm, nn, nk = m // bm, n // bn, k // bk

    operands = [a, b]
    in_specs = [
        pl.BlockSpec((bm, bk), lambda mi, ni, ki: (mi, ki)),
        pl.BlockSpec((bk, bn), lambda mi, ni, ki: (ki, ni)),
    ]
    if lhs_scale is not None:
        operands.append(lhs_scale)
        in_specs.append(pl.BlockSpec((bm, 1), lambda mi, ni, ki: (mi, 0)))
    if rhs_scale is not None:
        operands.append(rhs_scale)
        in_specs.append(pl.BlockSpec((bk, 1), lambda mi, ni, ki: (ki, 0)))
    if out_scale is not None:
        operands.append(out_scale)
        in_specs.append(pl.BlockSpec((bm, 1), lambda mi, ni, ki: (mi, 0)))
    if bias is not None:
        operands.append(bias.reshape(1, n))
        in_specs.append(pl.BlockSpec((1, bn), lambda mi, ni, ki: (0, ni)))

    def body(a_ref, b_ref, *rest):
        rest = list(rest)
        ls_ref = rest.pop(0) if lhs_scale is not None else None
        rs_ref = rest.pop(0) if rhs_scale is not None else None
        os_ref = rest.pop(0) if out_scale is not None else None
        bi_ref = rest.pop(0) if bias is not None else None
        o_ref = rest.pop(0)
        acc_ref = rest.pop(0) if nk > 1 else None

        av = a_ref[...]
        if ls_ref is not None:
            av = av * ls_ref[...]
        av = av.astype(jnp.bfloat16)
        bv = b_ref[...]
        if rs_ref is not None:
            bv = bv * rs_ref[...]
        bv = bv.astype(jnp.bfloat16)
        part = jnp.dot(av, bv, preferred_element_type=jnp.float32)

        def epilogue(r):
            if os_ref is not None:
                r = r * os_ref[...]
            if bi_ref is not None:
                r = r + bi_ref[...]
            if relu:
                r = jnp.maximum(r, 0.0)
            return r.astype(out_dtype)

        if nk == 1:
            o_ref[...] = epilogue(part)
        else:
            ki = pl.program_id(2)

            @pl.when(ki == 0)
            def _():
                acc_ref[...] = part

            @pl.when(ki > 0)
            def _():
                acc_ref[...] += part

            @pl.when(ki == nk - 1)
            def _():
                o_ref[...] = epilogue(acc_ref[...])

    scratch = [pltpu.VMEM((bm, bn), jnp.float32)] if nk > 1 else []
    return pl.pallas_call(
        body,
        grid=(nm, nn, nk),
        in_specs=in_specs,
        out_specs=pl.BlockSpec((bm, bn), lambda mi, ni, ki: (mi, ni)),
        out_shape=jax.ShapeDtypeStruct((m, n), out_dtype),
        scratch_shapes=scratch,
        compiler_params=pltpu.CompilerParams(
            dimension_semantics=("parallel", "parallel", "arbitrary")),
    )(*operands)


def _combine(p1, p2, p3, p4, p5, ws):
    br = 256
    ng = _N // br

    def body(w_ref, p1_ref, p2_ref, p3_ref, p4_ref, p5_ref,
             m1_ref, m2_ref, m3_ref, d1_ref, d2_ref, d3_ref):
        i = pl.program_id(0)
        rows = jax.lax.broadcasted_iota(jnp.int32, (br, _N), 0) + i * br
        cols = jax.lax.broadcasted_iota(jnp.int32, (br, _N), 1)
        diag = rows == cols
        pv = tuple(r[...].astype(jnp.float32)
                   for r in (p1_ref, p2_ref, p3_ref, p4_ref, p5_ref))
        for l, (m_ref, d_ref) in enumerate(
                ((m1_ref, d1_ref), (m2_ref, d2_ref), (m3_ref, d3_ref))):
            c = w_ref[l, 0]
            mv = jnp.where(diag, c, 0.0)
            for i_pow in range(_FILTER):
                c = c * w_ref[l, i_pow + 1]
                mv = mv + c * pv[i_pow]
            m_ref[...] = mv.astype(jnp.bfloat16)
            deg = jnp.sum((mv != 0.0).astype(jnp.float32), axis=1,
                          keepdims=True)
            d_ref[...] = jnp.where(
                deg > 0.0, jax.lax.rsqrt(jnp.maximum(deg, 1.0)), 0.0)

    pspec = pl.BlockSpec((br, _N), lambda i: (i, 0))
    dspec = pl.BlockSpec((br, 1), lambda i: (i, 0))
    mshape = jax.ShapeDtypeStruct((_N, _N), jnp.bfloat16)
    dshape = jax.ShapeDtypeStruct((_N, 1), jnp.float32)
    return pl.pallas_call(
        body,
        grid=(ng,),
        in_specs=[pl.BlockSpec(memory_space=pltpu.SMEM)] + [pspec] * 5,
        out_specs=[pspec] * 3 + [dspec] * 3,
        out_shape=[mshape] * 3 + [dshape] * 3,
        compiler_params=pltpu.CompilerParams(
            dimension_semantics=("arbitrary",)),
    )(ws, p1, p2, p3, p4, p5)


def kernel(x, edge_index, W1, b1, w1, W2, b2, w2, W3, b3, w3):
    A = _build_adj(edge_index).astype(jnp.bfloat16)
    W1 = W1.astype(jnp.bfloat16)
    W2 = W2.astype(jnp.bfloat16)
    W3 = W3.astype(jnp.bfloat16)

    p2 = _mm(A, A, 1024, 1024, 2048)
    p3 = _mm(p2, A, 1024, 1024, 2048)
    p4 = _mm(p3, A, 1024, 1024, 2048)
    p5 = _mm(p4, A, 1024, 1024, 2048)

    m1, m2, m3, d1, d2, d3 = _combine(A, p2, p3, p4, p5,
                                      jnp.stack([w1, w2, w3]))

    t = _mm(m1, x, 1024, 128, 2048, rhs_scale=d1, out_scale=d1)
    h = _mm(t, W1, 2048, 1280, 128, bias=b1, relu=True)

    u = _mm(h, W2, 512, 3200, 1280, out_scale=d2)
    h = _mm(m2, u, 512, 3200, 2048, out_scale=d2, bias=b2, relu=True)

    u = _mm(h, W3, 2048, 64, 3200, out_scale=d3)
    out = _mm(m3, u, 1024, 64, 2048, out_scale=d3, bias=b3, relu=True,
              out_dtype=jnp.float32)
    return out
